# baseline (speedup 1.0000x reference)
import jax
import jax.numpy as jnp
from jax.experimental import pallas as pl
from jax.experimental.pallas import tpu as pltpu

_IND_B = 1000


def _body(s_ref, ct_in, ind_in, t_ref, new_in, sv_in, ct_out, ind_out, sv_out):
    del sv_in
    rowi = s_ref[1]
    off = s_ref[3]
    ind = ind_in[...]
    ii = jax.lax.broadcasted_iota(jnp.int32, ind.shape, 2)
    hit = ii == off
    cur = jnp.sum(jnp.where(hit, ind, 0))
    ind_out[...] = ind + hit.astype(ind.dtype)
    blk = ct_in[...]
    ri = jax.lax.broadcasted_iota(jnp.int32, blk.shape, 0)
    ci = jax.lax.broadcasted_iota(jnp.int32, blk.shape, 1)
    ct_out[...] = jnp.where((ri == rowi) & (ci == cur), t_ref[0], blk)
    sv_out[...] = new_in[...]


def kernel(clicktimes, indices, idx, t, saved, new, save_index):
    n_det, max_clicks = clicktimes.shape
    n_save, batch, dim = saved.shape
    idx = jnp.asarray(idx, jnp.int32)
    sidx = jnp.asarray(save_index, jnp.int32)
    ind3 = indices.reshape(n_det // _IND_B, 1, _IND_B)
    s = jnp.stack([idx // 8, idx % 8, idx // _IND_B, idx % _IND_B, sidx])
    t_arr = jnp.reshape(t, (1,))

    grid_spec = pltpu.PrefetchScalarGridSpec(
        num_scalar_prefetch=1,
        grid=(1,),
        in_specs=[
            pl.BlockSpec((8, max_clicks), lambda i, s: (s[0], 0)),
            pl.BlockSpec((1, 1, _IND_B), lambda i, s: (s[2], 0, 0)),
            pl.BlockSpec(memory_space=pltpu.SMEM),
            pl.BlockSpec((1, batch, dim), lambda i, s: (s[4], 0, 0)),
            pl.BlockSpec((1, batch, dim), lambda i, s: (s[4], 0, 0)),
        ],
        out_specs=[
            pl.BlockSpec((8, max_clicks), lambda i, s: (s[0], 0)),
            pl.BlockSpec((1, 1, _IND_B), lambda i, s: (s[2], 0, 0)),
            pl.BlockSpec((1, batch, dim), lambda i, s: (s[4], 0, 0)),
        ],
    )
    ct_out, ind_out, sv_out = pl.pallas_call(
        _body,
        grid_spec=grid_spec,
        out_shape=[
            jax.ShapeDtypeStruct(clicktimes.shape, clicktimes.dtype),
            jax.ShapeDtypeStruct(ind3.shape, ind3.dtype),
            jax.ShapeDtypeStruct(saved.shape, saved.dtype),
        ],
        input_output_aliases={1: 0, 2: 1, 5: 2},
        compiler_params=pltpu.CompilerParams(
            dimension_semantics=("arbitrary",),
        ),
    )(s, clicktimes, ind3, t_arr, new, saved)
    return ct_out, ind_out.reshape(n_det), sv_out, save_index + 1


# speedup vs baseline: 4.3823x; 4.3823x over previous
import jax
import jax.numpy as jnp
from jax.experimental import pallas as pl
from jax.experimental.pallas import tpu as pltpu

_IND_R = 100
_LANES = 128


def _body(s_ref, ct_in, ind_in, t_ref, new_in, sv_in, ct_out, ind_out, sv_out):
    del sv_in
    lane = s_ref[1]
    idx = s_ref[2]
    ind = ind_in[...]
    ri = jax.lax.broadcasted_iota(jnp.int32, ind.shape, 0)
    ci = jax.lax.broadcasted_iota(jnp.int32, ind.shape, 1)
    flat = ri * (ind.shape[1]) + ci
    hit = flat == idx
    cur = jnp.sum(jnp.where(hit, ind, 0))
    ind_out[...] = ind + hit.astype(ind.dtype)
    blk = ct_in[...]
    cri = jax.lax.broadcasted_iota(jnp.int32, blk.shape, 0)
    cci = jax.lax.broadcasted_iota(jnp.int32, blk.shape, 1)
    ct_out[...] = jnp.where((cri == cur) & (cci == lane), t_ref[0], blk)
    sv_out[...] = new_in[...]


def kernel(clicktimes, indices, idx, t, saved, new, save_index):
    n_det, max_clicks = clicktimes.shape
    n_save, batch, dim = saved.shape
    idx = jnp.asarray(idx, jnp.int32)
    sidx = jnp.asarray(save_index, jnp.int32)
    ct_t = clicktimes.T
    sv_t = jnp.transpose(saved, (0, 2, 1))
    new_t = jnp.transpose(new, (0, 2, 1))
    ind2 = indices.reshape(_IND_R, n_det // _IND_R)
    s = jnp.stack([idx // _LANES, idx % _LANES, idx, sidx])
    t_arr = jnp.reshape(t, (1,))

    grid_spec = pltpu.PrefetchScalarGridSpec(
        num_scalar_prefetch=1,
        grid=(1,),
        in_specs=[
            pl.BlockSpec((max_clicks, _LANES), lambda i, s: (0, s[0])),
            pl.BlockSpec(ind2.shape, lambda i, s: (0, 0)),
            pl.BlockSpec(memory_space=pltpu.SMEM),
            pl.BlockSpec((1, dim, batch), lambda i, s: (s[3], 0, 0)),
            pl.BlockSpec((1, 8, _LANES), lambda i, s: (s[3], 0, 0)),
        ],
        out_specs=[
            pl.BlockSpec((max_clicks, _LANES), lambda i, s: (0, s[0])),
            pl.BlockSpec(ind2.shape, lambda i, s: (0, 0)),
            pl.BlockSpec((1, dim, batch), lambda i, s: (s[3], 0, 0)),
        ],
    )
    ct_out, ind_out, sv_out = pl.pallas_call(
        _body,
        grid_spec=grid_spec,
        out_shape=[
            jax.ShapeDtypeStruct(ct_t.shape, ct_t.dtype),
            jax.ShapeDtypeStruct(ind2.shape, ind2.dtype),
            jax.ShapeDtypeStruct(sv_t.shape, sv_t.dtype),
        ],
        input_output_aliases={1: 0, 5: 2},
        compiler_params=pltpu.CompilerParams(
            dimension_semantics=("arbitrary",),
        ),
    )(s, ct_t, ind2, t_arr, new_t, sv_t)
    return (
        ct_out.T,
        ind_out.reshape(n_det),
        jnp.transpose(sv_out, (0, 2, 1)),
        save_index + 1,
    )
